# baseline (speedup 1.0000x reference)
import functools

import jax
import jax.numpy as jnp
from jax.experimental import pallas as pl
from jax.experimental.pallas import tpu as pltpu

_B, _N, _F, _H, _NH, _NHID, _OUT = 4, 1024, 128, 128, 8, 16, 64
_ALPHA = 0.02
_NEG = -9e15


def _gmodel_body(x_ref, sup_ref, wg_ref, bg_ref, lng_ref, lnb_ref,
                 w2_ref, asrc_ref, adstT_ref, wout_ref,
                 aos_ref, aod_ref, out_ref):
    f32 = jnp.float32
    x = x_ref[0]
    sup = sup_ref[0]

    xw = jnp.dot(x, wg_ref[...], preferred_element_type=f32)
    h = jnp.dot(sup, xw, preferred_element_type=f32) + bg_ref[...]
    h = jnp.maximum(h, 0.0)

    mu = jnp.mean(h, axis=-1, keepdims=True)
    hc = h - mu
    var = jnp.mean(hc * hc, axis=-1, keepdims=True)
    h = hc * jax.lax.rsqrt(var + 1e-5) * lng_ref[...] + lnb_ref[...]

    Wh = jnp.dot(h, w2_ref[...], preferred_element_type=f32)
    S = jnp.dot(Wh, asrc_ref[...], preferred_element_type=f32)
    Drow = jax.lax.dot_general(adstT_ref[...], Wh,
                               (((1,), (1,)), ((), ())),
                               preferred_element_type=f32)

    mask01 = jnp.where(sup > 0.0, 1.0, 0.0)
    maxd = jnp.max(Drow, axis=1, keepdims=True)
    t0 = S + maxd[:, 0][None, :]
    m = jnp.where(t0 >= 0, t0, _ALPHA * t0)
    E = jnp.exp(S - m)
    G = jnp.exp(_ALPHA * S - m)
    F = jnp.exp(Drow)
    Hh = jnp.exp(_ALPHA * Drow)

    bf16 = jnp.bfloat16
    mask16 = mask01.astype(bf16)
    E16, G16 = E.astype(bf16), G.astype(bf16)
    F16, H16 = F.astype(bf16), Hh.astype(bf16)
    ones_col = jnp.ones((_N, 1), f32)
    oes = []
    for hh in range(_NH):
        p = jnp.maximum(E16[:, hh][:, None] * F16[hh, :][None, :],
                        G16[:, hh][:, None] * H16[hh, :][None, :]) * mask16
        rhs = jnp.concatenate(
            [Wh[:, hh * _NHID:(hh + 1) * _NHID], ones_col], axis=1)
        oes.append(jnp.dot(p, rhs.astype(bf16), preferred_element_type=f32))
    num = jnp.concatenate([oe[:, :_NHID] for oe in oes], axis=1)
    den8 = jnp.concatenate(
        [oe[:, _NHID:_NHID + 1] for oe in oes], axis=1)
    blk = (jax.lax.broadcasted_iota(jnp.int32, (_NH, _H), 1) // _NHID
           == jax.lax.broadcasted_iota(jnp.int32, (_NH, _H), 0)
           ).astype(f32)
    cat = num / jnp.dot(den8, blk, preferred_element_type=f32)
    cat = jnp.where(cat > 0, cat,
                    jnp.exp(jnp.minimum(cat, 0.0)) - 1.0)

    Who = jnp.dot(cat, wout_ref[...], preferred_element_type=f32)
    s2 = jnp.dot(Who, aos_ref[...], preferred_element_type=f32)
    d2 = jax.lax.dot_general(aod_ref[...], Who,
                             (((0,), (1,)), ((), ())),
                             preferred_element_type=f32)
    t2 = s2 + jnp.max(d2, axis=1, keepdims=True)
    m2 = jnp.where(t2 >= 0, t2, _ALPHA * t2)
    E2 = jnp.exp(s2 - m2).astype(bf16)
    G2 = jnp.exp(_ALPHA * s2 - m2).astype(bf16)
    F2 = jnp.exp(d2).astype(bf16)
    H2a = jnp.exp(_ALPHA * d2).astype(bf16)
    p2 = jnp.maximum(E2 * F2, G2 * H2a) * mask16
    rhs2 = jnp.concatenate([Who, ones_col], axis=1)
    oe2 = jnp.dot(p2, rhs2.astype(bf16), preferred_element_type=f32)
    o2 = oe2[:, :_OUT] / oe2[:, _OUT:_OUT + 1]
    o2 = jnp.where(o2 > 0, o2, jnp.exp(jnp.minimum(o2, 0.0)) - 1.0)

    out_ref[0] = (jnp.mean(o2, axis=0, keepdims=True)
                  + jnp.max(o2, axis=0, keepdims=True))


@functools.partial(jax.jit, static_argnames=("interpret",))
def _run(x, support, W_gcn, bg, lng, lnb, W2, Asrc, AdstT, W_out,
         aos, aod, interpret=False):
    full = lambda shape: pl.BlockSpec(shape, lambda b: (0,) * len(shape))
    pooled3 = pl.pallas_call(
        _gmodel_body,
        grid=(_B,),
        in_specs=[
            pl.BlockSpec((1, _N, _F), lambda b: (b, 0, 0)),
            pl.BlockSpec((1, _N, _N), lambda b: (b, 0, 0)),
            full((_F, _H)),
            full((1, _H)),
            full((1, _H)),
            full((1, _H)),
            full((_H, _NH * _NHID)),
            full((_NH * _NHID, _NH)),
            full((_NH, _NH * _NHID)),
            full((_H, _OUT)),
            full((_OUT, 1)),
            full((_OUT, 1)),
        ],
        out_specs=pl.BlockSpec((1, 1, _OUT), lambda b: (b, 0, 0)),
        out_shape=jax.ShapeDtypeStruct((_B, 1, _OUT), jnp.float32),
        compiler_params=pltpu.CompilerParams(
            dimension_semantics=("arbitrary",)),
        interpret=interpret,
    )(x, support, W_gcn, bg, lng, lnb, W2, Asrc, AdstT, W_out, aos, aod)
    return pooled3.reshape(_B, _OUT)


def kernel(x, support, mask, W_gcn, b_gcn, ln_g, ln_b, W_att, a_src, a_dst,
           W_out, ao_src, ao_dst):
    H2 = _NH * _NHID
    W2 = jnp.transpose(W_att, (1, 0, 2)).reshape(_H, H2)
    head_of = jnp.arange(H2) // _NHID
    blkdiag = (head_of[:, None] == jnp.arange(_NH)[None, :]).astype(
        jnp.float32)
    Asrc = blkdiag * a_src.reshape(H2)[:, None]
    AdstT = blkdiag.T * a_dst.reshape(H2)[None, :]
    return _run(x, support, W_gcn,
                b_gcn.reshape(1, _H), ln_g.reshape(1, _H), ln_b.reshape(1, _H),
                W2, Asrc, AdstT, W_out,
                ao_src.reshape(_OUT, 1), ao_dst.reshape(_OUT, 1))


# speedup vs baseline: 1.0156x; 1.0156x over previous
import functools

import jax
import jax.numpy as jnp
from jax.experimental import pallas as pl
from jax.experimental.pallas import tpu as pltpu

_B, _N, _F, _H, _NH, _NHID, _OUT = 4, 1024, 128, 128, 8, 16, 64
_ALPHA = 0.02
_NEG = -9e15


def _gmodel_body(x_ref, sup_ref, wg_ref, bg_ref, lng_ref, lnb_ref,
                 w2_ref, asrc_ref, adstT_ref, wout_ref,
                 aos_ref, aod_ref, out_ref):
    f32 = jnp.float32
    x = x_ref[0]
    sup = sup_ref[0]

    xw = jnp.dot(x, wg_ref[...], preferred_element_type=f32)
    h = jnp.dot(sup, xw, preferred_element_type=f32) + bg_ref[...]
    h = jnp.maximum(h, 0.0)

    mu = jnp.mean(h, axis=-1, keepdims=True)
    hc = h - mu
    var = jnp.mean(hc * hc, axis=-1, keepdims=True)
    h = hc * jax.lax.rsqrt(var + 1e-5) * lng_ref[...] + lnb_ref[...]

    Wh = jnp.dot(h, w2_ref[...], preferred_element_type=f32)
    S = jnp.dot(Wh, asrc_ref[...], preferred_element_type=f32)
    Drow = jax.lax.dot_general(adstT_ref[...], Wh,
                               (((1,), (1,)), ((), ())),
                               preferred_element_type=f32)

    maxd = jnp.max(Drow, axis=1, keepdims=True)
    t0 = S + maxd[:, 0][None, :]
    m = jnp.where(t0 >= 0, t0, _ALPHA * t0)
    E = jnp.exp(S - m)
    G = jnp.exp(_ALPHA * S - m)
    F = jnp.exp(Drow)
    Hh = jnp.exp(_ALPHA * Drow)

    bf16 = jnp.bfloat16
    mask16 = jnp.where(sup > 0.0, 1.0, 0.0).astype(bf16)
    E16, G16 = E.astype(bf16), G.astype(bf16)
    F16, H16 = F.astype(bf16), Hh.astype(bf16)
    ones_col = jnp.ones((_N, 1), f32)
    outs = []
    for hh in range(_NH):
        p = jnp.maximum(E16[:, hh][:, None] * F16[hh, :][None, :],
                        G16[:, hh][:, None] * H16[hh, :][None, :]) * mask16
        rhs = jnp.concatenate(
            [Wh[:, hh * _NHID:(hh + 1) * _NHID], ones_col], axis=1)
        oe = jnp.dot(p, rhs.astype(bf16), preferred_element_type=f32)
        o = oe[:, :_NHID] / oe[:, _NHID:_NHID + 1]
        outs.append(o)
    cat = jnp.concatenate(outs, axis=1)
    cat = jnp.where(cat > 0, cat,
                    jnp.exp(jnp.minimum(cat, 0.0)) - 1.0)

    Who = jnp.dot(cat, wout_ref[...], preferred_element_type=f32)
    s2 = jnp.dot(Who, aos_ref[...], preferred_element_type=f32)
    d2 = jax.lax.dot_general(aod_ref[...], Who,
                             (((0,), (1,)), ((), ())),
                             preferred_element_type=f32)
    t2 = s2 + jnp.max(d2, axis=1, keepdims=True)
    m2 = jnp.where(t2 >= 0, t2, _ALPHA * t2)
    E2 = jnp.exp(s2 - m2).astype(bf16)
    G2 = jnp.exp(_ALPHA * s2 - m2).astype(bf16)
    F2 = jnp.exp(d2).astype(bf16)
    H2a = jnp.exp(_ALPHA * d2).astype(bf16)
    p2 = jnp.maximum(E2 * F2, G2 * H2a) * mask16
    rhs2 = jnp.concatenate([Who, ones_col], axis=1)
    oe2 = jnp.dot(p2, rhs2.astype(bf16), preferred_element_type=f32)
    o2 = oe2[:, :_OUT] / oe2[:, _OUT:_OUT + 1]
    o2 = jnp.where(o2 > 0, o2, jnp.exp(jnp.minimum(o2, 0.0)) - 1.0)

    out_ref[0] = (jnp.mean(o2, axis=0, keepdims=True)
                  + jnp.max(o2, axis=0, keepdims=True))


@functools.partial(jax.jit, static_argnames=("interpret",))
def _run(x, support, W_gcn, bg, lng, lnb, W2, Asrc, AdstT, W_out,
         aos, aod, interpret=False):
    full = lambda shape: pl.BlockSpec(shape, lambda b: (0,) * len(shape))
    pooled3 = pl.pallas_call(
        _gmodel_body,
        grid=(_B,),
        in_specs=[
            pl.BlockSpec((1, _N, _F), lambda b: (b, 0, 0)),
            pl.BlockSpec((1, _N, _N), lambda b: (b, 0, 0)),
            full((_F, _H)),
            full((1, _H)),
            full((1, _H)),
            full((1, _H)),
            full((_H, _NH * _NHID)),
            full((_NH * _NHID, _NH)),
            full((_NH, _NH * _NHID)),
            full((_H, _OUT)),
            full((_OUT, 1)),
            full((_OUT, 1)),
        ],
        out_specs=pl.BlockSpec((1, 1, _OUT), lambda b: (b, 0, 0)),
        out_shape=jax.ShapeDtypeStruct((_B, 1, _OUT), jnp.float32),
        compiler_params=pltpu.CompilerParams(
            dimension_semantics=("arbitrary",)),
        interpret=interpret,
    )(x, support, W_gcn, bg, lng, lnb, W2, Asrc, AdstT, W_out, aos, aod)
    return pooled3.reshape(_B, _OUT)


def kernel(x, support, mask, W_gcn, b_gcn, ln_g, ln_b, W_att, a_src, a_dst,
           W_out, ao_src, ao_dst):
    H2 = _NH * _NHID
    W2 = jnp.transpose(W_att, (1, 0, 2)).reshape(_H, H2)
    head_of = jnp.arange(H2) // _NHID
    blkdiag = (head_of[:, None] == jnp.arange(_NH)[None, :]).astype(
        jnp.float32)
    Asrc = blkdiag * a_src.reshape(H2)[:, None]
    AdstT = blkdiag.T * a_dst.reshape(H2)[None, :]
    return _run(x, support, W_gcn,
                b_gcn.reshape(1, _H), ln_g.reshape(1, _H), ln_b.reshape(1, _H),
                W2, Asrc, AdstT, W_out,
                ao_src.reshape(_OUT, 1), ao_dst.reshape(_OUT, 1))
